# Initial kernel scaffold; baseline (speedup 1.0000x reference)
import jax
import jax.numpy as jnp
from jax.experimental import pallas as pl


def kernel(x, W1, b1, Va, ba, wa, bwa, Wp, bp):
    raise NotImplementedError("write your pallas kernel here")


# speedup vs baseline: 2.6839x; 2.6839x over previous
import functools

import jax
import jax.numpy as jnp
from jax.experimental import pallas as pl
from jax.experimental.pallas import tpu as pltpu

N = 8192
D_IN = 1024
D = 512
DA = 128
K_MASK = 819
R = 512
T = N // R


def _avg_floor(lo, hi):
    return (lo >> 1) + (hi >> 1) + (lo & hi & 1)


def _mhim_kernel(x_ref, w1_ref, b1_ref, va_ref, ba_ref, wa_ref, bwa_ref,
                 wp_ref, bp_ref, out_ref, feat_scr, scores_scr):
    i = pl.program_id(0)

    xt = x_ref[...]
    feat = jnp.maximum(
        jax.lax.dot_general(xt, w1_ref[...], (((1,), (0,)), ((), ()))) +
        b1_ref[...], 0.0)
    feat_scr[pl.ds(i * R, R), :] = feat
    h = jax.nn.gelu(
        jax.lax.dot_general(feat, va_ref[...], (((1,), (0,)), ((), ()))) +
        ba_ref[...])
    s = jax.lax.dot_general(wa_ref[...], h, (((0,), (1,)), ((), ()))) + \
        bwa_ref[0, 0]
    scores_scr[pl.ds(i, 1), :] = s

    @pl.when(i == T - 1)
    def _final():
        S = scores_scr[...]
        bits = jax.lax.bitcast_convert_type(S, jnp.int32)
        m = bits ^ ((bits >> 31) & jnp.int32(0x7FFFFFFF))

        def vb(_, c):
            lo, hi = c
            mid = _avg_floor(lo, hi)
            cnt = jnp.sum((m > mid).astype(jnp.int32))
            big = cnt >= K_MASK
            return (jnp.where(big, mid + 1, lo), jnp.where(big, hi, mid))
        lo, hi = jax.lax.fori_loop(0, 32, vb, (jnp.min(m), jnp.max(m)))
        vstar = lo

        c_gt = jnp.sum((m > vstar).astype(jnp.int32))
        t_ties = K_MASK - c_gt
        eq = m == vstar
        idx = (jax.lax.broadcasted_iota(jnp.int32, (T, R), 0) * R +
               jax.lax.broadcasted_iota(jnp.int32, (T, R), 1))

        def ib(_, c):
            lo2, hi2 = c
            mid = lo2 + ((hi2 - lo2 + 1) >> 1)
            cnt = jnp.sum((eq & (idx >= mid)).astype(jnp.int32))
            ok = cnt >= t_ties
            return (jnp.where(ok, mid, lo2), jnp.where(ok, hi2, mid - 1))
        istar, _ = jax.lax.fori_loop(
            0, 14, ib, (jnp.int32(0), jnp.int32(N)))

        masked = (m > vstar) | (eq & (idx >= istar))
        keep = jnp.logical_not(masked)

        smax = jnp.max(jnp.where(keep, S, -jnp.inf))
        e = jnp.where(keep, jnp.exp(S - smax), 0.0)
        w = e / jnp.sum(e)
        scores_scr[...] = w

        def pb(p, acc):
            wp = scores_scr[pl.ds(p, 1), :]
            fp = feat_scr[pl.ds(p * R, R), :]
            return acc + jax.lax.dot_general(
                wp, fp, (((1,), (0,)), ((), ())))
        bag = jax.lax.fori_loop(
            0, T, pb, jnp.zeros((1, D), dtype=jnp.float32))

        out_ref[...] = jax.lax.dot_general(
            bag, wp_ref[...], (((1,), (0,)), ((), ()))) + bp_ref[...]


@functools.partial(jax.jit, static_argnames=())
def kernel(x, W1, b1, Va, ba, wa, bwa, Wp, bp):
    x2 = x.reshape(N, D_IN)
    out = pl.pallas_call(
        _mhim_kernel,
        grid=(T,),
        in_specs=[
            pl.BlockSpec((R, D_IN), lambda i: (i, 0)),
            pl.BlockSpec((D_IN, D), lambda i: (0, 0)),
            pl.BlockSpec((1, D), lambda i: (0, 0)),
            pl.BlockSpec((D, DA), lambda i: (0, 0)),
            pl.BlockSpec((1, DA), lambda i: (0, 0)),
            pl.BlockSpec((DA, 1), lambda i: (0, 0)),
            pl.BlockSpec((1, 1), lambda i: (0, 0)),
            pl.BlockSpec((D, 2), lambda i: (0, 0)),
            pl.BlockSpec((1, 2), lambda i: (0, 0)),
        ],
        out_specs=pl.BlockSpec((1, 2), lambda i: (0, 0)),
        out_shape=jax.ShapeDtypeStruct((1, 2), jnp.float32),
        scratch_shapes=[
            pltpu.VMEM((N, D), jnp.float32),
            pltpu.VMEM((T, R), jnp.float32),
        ],
    )(x2, W1, b1.reshape(1, D), Va, ba.reshape(1, DA), wa,
      bwa.reshape(1, 1), Wp, bp.reshape(1, 2))
    return out
